# Initial kernel scaffold; baseline (speedup 1.0000x reference)
import jax
import jax.numpy as jnp
from jax.experimental import pallas as pl


def kernel(self_att_layer_loga, cross_att_layer_loga, ffn_layer_loga):
    raise NotImplementedError("write your pallas kernel here")


# speedup vs baseline: 8.6759x; 8.6759x over previous
import numpy as np
import jax
import jax.numpy as jnp
from jax.experimental import pallas as pl

_EPS = 1e-6
_LIMIT_A = -0.1
_LIMIT_B = 1.1
_BETA = 2.0 / 3.0
_XN = (0.0 - _LIMIT_A) / (_LIMIT_B - _LIMIT_A)
_C = float(np.log(_XN / (1.0 - _XN))) * _BETA


def _tc_body(x_ref, o_ref):
    x = x_ref[...]
    n = x.shape[1]

    s = jax.nn.sigmoid(_C - x)
    s = jnp.clip(s, _EPS, 1.0 - _EPS)
    enz = np.float32(n) - jnp.sum(1.0 - s, axis=1, keepdims=True)
    k = jnp.round(enz).astype(jnp.int32)

    xc = jnp.where(x == 0.0, 0.0, x)
    u = jax.lax.bitcast_convert_type(xc, jnp.uint32)
    flip = jnp.where((u >> 31) != 0, jnp.uint32(0xFFFFFFFF), jnp.uint32(0x80000000))
    ukey = u ^ flip

    count_lt = jnp.zeros_like(k)
    prefix = jnp.zeros((3, 1), jnp.uint32)
    for r in range(32):
        bit = jnp.uint32(1 << (31 - r))
        hm = jnp.uint32((0xFFFFFFFF << (32 - r)) & 0xFFFFFFFF) if r else jnp.uint32(0)
        active = (ukey & hm) == prefix
        m0 = active & ((ukey & bit) == 0)
        cnt0 = jnp.sum(m0.astype(jnp.int32), axis=1, keepdims=True)
        take_low = (count_lt + cnt0) >= k
        prefix = jnp.where(take_low, prefix, prefix | bit)
        count_lt = jnp.where(take_low, count_lt, count_lt + cnt0)
    t_key = prefix

    tie_budget = k - count_lt
    eq = ukey == t_key
    idx = jax.lax.broadcasted_iota(jnp.int32, x.shape, 1)
    prefix2 = jnp.zeros_like(k)
    cnt_lt2 = jnp.zeros_like(k)
    for r in range(13):
        bit = np.int32(1 << (12 - r))
        hm2 = np.int32(((0x1FFF >> (13 - r)) << (13 - r)))
        active2 = eq & ((idx & hm2) == prefix2)
        m0 = active2 & ((idx & bit) == 0)
        cnt0 = jnp.sum(m0.astype(jnp.int32), axis=1, keepdims=True)
        take_low = (cnt_lt2 + cnt0) >= tie_budget
        prefix2 = jnp.where(take_low, prefix2, prefix2 | bit)
        cnt_lt2 = jnp.where(take_low, cnt_lt2, cnt_lt2 + cnt0)
    i_cut = prefix2

    ikey = jax.lax.bitcast_convert_type(ukey ^ jnp.uint32(0x80000000), jnp.int32)
    it_key = jax.lax.bitcast_convert_type(t_key ^ jnp.uint32(0x80000000), jnp.int32)
    zero = (ikey < it_key) | (eq & (idx <= i_cut))
    o_ref[...] = jnp.where(zero, 0.0, 1.0).astype(jnp.float32)


def _run(x, interpret=False):
    return pl.pallas_call(
        _tc_body,
        out_shape=jax.ShapeDtypeStruct((3, 8192), jnp.float32),
        interpret=interpret,
    )(x)


def kernel(self_att_layer_loga, cross_att_layer_loga, ffn_layer_loga):
    x = jnp.stack([self_att_layer_loga, cross_att_layer_loga, ffn_layer_loga])
    out = _run(x)
    return (out[0], out[1], out[2])
